# Initial kernel scaffold; baseline (speedup 1.0000x reference)
import jax
import jax.numpy as jnp
from jax.experimental import pallas as pl


def kernel(t_emb, c_emb, inputs, context):
    raise NotImplementedError("write your pallas kernel here")


# speedup vs baseline: 10.4617x; 10.4617x over previous
import functools

import jax
import jax.numpy as jnp
import numpy as np
from jax import lax
from jax.experimental import pallas as pl
from jax.experimental.pallas import tpu as pltpu
from jax.experimental.pallas import tpu_sc as plsc

VOCAB = 100001
DIM = 64
BATCH = 16384
CTX = 51

NUM_CORES = 2
NUM_SUBCORES = 16
NUM_WORKERS = NUM_CORES * NUM_SUBCORES
RPW = BATCH // NUM_WORKERS
TCH = 128
NTCH = RPW // TCH
OUTP = 64


def _dot_row(trow, crows, out_v, r):
    t = [trow[pl.ds(k * 16, 16)] for k in range(4)]
    lane = lax.iota(jnp.int32, 16)
    perms = [lane ^ m for m in (8, 4, 2, 1)]
    row_base = r * CTX + lane
    for cc in range((CTX + 15) // 16):
        width = min(16, CTX - cc * 16)
        res = jnp.zeros((16,), jnp.float32)
        for cl in range(width):
            c = cc * 16 + cl
            p = t[0] * crows[c, pl.ds(0, 16)]
            for k in range(1, 4):
                p = p + t[k] * crows[c, pl.ds(k * 16, 16)]
            for perm in perms:
                p = p + p.at[perm].get(mode="promise_in_bounds")
            res = jnp.where(lane == cl, p, res)
        mask = (lane < width) if width < 16 else None
        plsc.store_scatter(out_v, [row_base + cc * 16], res, mask=mask)


def _body(t_emb, c_emb, inputs, context, out, t_idx_v, trows_v, ctx_idx_v,
          crows0, crows1, out_v, sem_t, sem0, sem1):
    cid = lax.axis_index("c")
    sid = lax.axis_index("s")
    wid = sid * NUM_CORES + cid
    base = wid * RPW

    for j in range(NTCH):
        pltpu.sync_copy(inputs.at[pl.ds(base + j * TCH, TCH)], t_idx_v.at[j])
    for j in range(NTCH):
        pltpu.make_async_copy(
            t_emb.at[t_idx_v.at[j]], trows_v.at[pl.ds(j * TCH, TCH)], sem_t
        ).start()
    pltpu.sync_copy(context.at[pl.ds(base, RPW)], ctx_idx_v)
    for j in range(NTCH):
        pltpu.make_async_copy(
            t_emb.at[t_idx_v.at[j]], trows_v.at[pl.ds(j * TCH, TCH)], sem_t
        ).wait()

    def ctx_copy(r, buf, sem):
        return pltpu.make_async_copy(c_emb.at[ctx_idx_v.at[r]], buf, sem)

    ctx_copy(0, crows0, sem0).start()
    ctx_copy(1, crows1, sem1).start()

    def pair(i, _):
        r = 2 * i
        ctx_copy(r, crows0, sem0).wait()
        _dot_row(trows_v.at[r], crows0, out_v, r)
        ctx_copy(r + 2, crows0, sem0).start()
        ctx_copy(r + 1, crows1, sem1).wait()
        _dot_row(trows_v.at[r + 1], crows1, out_v, r + 1)
        ctx_copy(r + 3, crows1, sem1).start()
        return 0

    lax.fori_loop(0, RPW // 2 - 1, pair, 0)

    r = RPW - 2
    ctx_copy(r, crows0, sem0).wait()
    _dot_row(trows_v.at[r], crows0, out_v, r)
    ctx_copy(r + 1, crows1, sem1).wait()
    _dot_row(trows_v.at[r + 1], crows1, out_v, r + 1)

    pltpu.sync_copy(out_v, out.at[pl.ds(base * CTX, RPW * CTX)])


@jax.jit
def kernel(t_emb, c_emb, inputs, context):
    mesh = plsc.VectorSubcoreMesh(core_axis_name="c", subcore_axis_name="s")
    k = pl.kernel(
        _body,
        out_type=jax.ShapeDtypeStruct((BATCH * CTX,), jnp.float32),
        mesh=mesh,
        compiler_params=pltpu.CompilerParams(
            use_tc_tiling_on_sc=False, needs_layout_passes=False
        ),
        scratch_types=[
            pltpu.VMEM((NTCH, TCH), jnp.int32),
            pltpu.VMEM((RPW, DIM), jnp.float32),
            pltpu.VMEM((RPW, CTX), jnp.int32),
            pltpu.VMEM((CTX, DIM), jnp.float32),
            pltpu.VMEM((CTX, DIM), jnp.float32),
            pltpu.VMEM((RPW * CTX,), jnp.float32),
            pltpu.SemaphoreType.DMA,
            pltpu.SemaphoreType.DMA,
            pltpu.SemaphoreType.DMA,
        ],
    )
    y = k(t_emb, c_emb, inputs.astype(jnp.int32), context.astype(jnp.int32))
    return y.reshape(BATCH, CTX)
